# Initial kernel scaffold; baseline (speedup 1.0000x reference)
import jax
import jax.numpy as jnp
from jax.experimental import pallas as pl


def kernel(inp, weight):
    raise NotImplementedError("write your pallas kernel here")


# speedup vs baseline: 1.6848x; 1.6848x over previous
import functools

import jax
import jax.numpy as jnp
from jax import lax
from jax.experimental import pallas as pl
from jax.experimental.pallas import tpu as pltpu
from jax.experimental.pallas import tpu_sc as plsc

NC = 2
NS = 16
NW = NC * NS

VOCAB = 1000000
EMBED_DIM = 64
CHUNK = 128


def _make_gather(total_rows: int):
    assert total_rows % (NW * CHUNK) == 0
    per_w = total_rows // NW
    n_chunks = per_w // CHUNK
    mesh = plsc.VectorSubcoreMesh(core_axis_name="c", subcore_axis_name="s")

    @functools.partial(
        pl.kernel,
        out_type=jax.ShapeDtypeStruct((total_rows, EMBED_DIM), jnp.float32),
        mesh=mesh,
        compiler_params=pltpu.CompilerParams(use_tc_tiling_on_sc=False),
        scratch_types=[
            pltpu.VMEM((n_chunks, CHUNK), jnp.int32),
            pltpu.VMEM((CHUNK, EMBED_DIM), jnp.float32),
            pltpu.SemaphoreType.DMA,
        ],
    )
    def gather_kernel(idx_hbm, table_hbm, out_hbm, idx_v, rows_v, sem):
        wid = lax.axis_index("s") * NC + lax.axis_index("c")
        base = wid * per_w
        pltpu.sync_copy(idx_hbm.at[wid], idx_v)

        @pl.loop(0, n_chunks)
        def _chunk(j):
            pltpu.async_copy(table_hbm.at[idx_v.at[j]], rows_v, sem).wait()
            pltpu.sync_copy(rows_v, out_hbm.at[pl.ds(base + j * CHUNK, CHUNK)])

    return gather_kernel


def kernel(inp, weight):
    batch, hist = inp.shape
    total = batch * hist
    idx = inp.astype(jnp.int32).reshape(NW, total // (NW * CHUNK), CHUNK)
    out = _make_gather(total)(idx, weight)
    return out.reshape(batch, hist, EMBED_DIM)


# speedup vs baseline: 1.8717x; 1.1109x over previous
import functools

import jax
import jax.numpy as jnp
from jax import lax
from jax.experimental import pallas as pl
from jax.experimental.pallas import tpu as pltpu
from jax.experimental.pallas import tpu_sc as plsc

NC = 2
NS = 16
NW = NC * NS

VOCAB = 1000000
EMBED_DIM = 64
CHUNK = 128


QPG = 4
GROUP = CHUNK * QPG


def _make_gather(total_rows: int):
    assert total_rows % (NW * GROUP) == 0
    per_w = total_rows // NW
    n_chunks = per_w // CHUNK
    n_groups = per_w // GROUP
    mesh = plsc.VectorSubcoreMesh(core_axis_name="c", subcore_axis_name="s")

    @functools.partial(
        pl.kernel,
        out_type=jax.ShapeDtypeStruct((total_rows, EMBED_DIM), jnp.float32),
        mesh=mesh,
        compiler_params=pltpu.CompilerParams(use_tc_tiling_on_sc=False),
        scratch_types=[
            pltpu.VMEM((n_chunks, CHUNK), jnp.int32),
            pltpu.VMEM((GROUP, EMBED_DIM), jnp.float32),
            pltpu.VMEM((GROUP, EMBED_DIM), jnp.float32),
            pltpu.SemaphoreType.DMA,
            pltpu.SemaphoreType.DMA,
            pltpu.SemaphoreType.DMA,
        ],
    )
    def gather_kernel(idx_hbm, table_hbm, out_hbm, idx_v, buf0, buf1, gsem,
                      osem0, osem1):
        wid = lax.axis_index("s") * NC + lax.axis_index("c")
        base = wid * per_w
        pltpu.sync_copy(idx_hbm.at[wid], idx_v)
        bufs = (buf0, buf1)
        osems = (osem0, osem1)

        @pl.loop(0, n_groups)
        def _group(g):
            for b in range(2):
                @pl.when(g % 2 == b)
                def _():
                    buf, osem = bufs[b], osems[b]
                    @pl.when(g >= 2)
                    def _():
                        pltpu.make_async_copy(
                            buf, out_hbm.at[pl.ds(base, GROUP)], osem
                        ).wait()
                    descs = [
                        pltpu.async_copy(
                            table_hbm.at[idx_v.at[g * QPG + q]],
                            buf.at[pl.ds(q * CHUNK, CHUNK)],
                            gsem,
                        )
                        for q in range(QPG)
                    ]
                    for d in descs:
                        d.wait()
                    pltpu.async_copy(
                        buf, out_hbm.at[pl.ds(base + g * GROUP, GROUP)], osem
                    )

        for b in range(2):
            pltpu.make_async_copy(
                bufs[b], out_hbm.at[pl.ds(base, GROUP)], osems[b]
            ).wait()

    return gather_kernel


def kernel(inp, weight):
    batch, hist = inp.shape
    total = batch * hist
    idx = inp.astype(jnp.int32).reshape(NW, total // (NW * CHUNK), CHUNK)
    out = _make_gather(total)(idx, weight)
    return out.reshape(batch, hist, EMBED_DIM)
